# baseline (speedup 1.0000x reference)
import jax
import jax.numpy as jnp
from jax.experimental import pallas as pl
from jax.experimental.pallas import tpu as pltpu
from functools import partial

TILE = 2048


def _fused_body(x_ref, seg_ref, w1_ref, b1_ref, w2_ref, b2_ref,
                wm_ref, bm_ref, wv_ref, bv_ref, eps_ref,
                out_ref, acc_ref, *, num_tiles, num_segments):
    i = pl.program_id(0)

    @pl.when(i == 0)
    def _init():
        acc_ref[...] = jnp.zeros_like(acc_ref)

    x = x_ref[...]
    h = jnp.dot(x, w1_ref[...], preferred_element_type=jnp.float32)
    h = jnp.maximum(h + b1_ref[...], 0.0).astype(jnp.bfloat16)
    h = jnp.dot(h, w2_ref[...], preferred_element_type=jnp.float32)
    h = jnp.maximum(h + b2_ref[...], 0.0).astype(jnp.bfloat16)

    seg = seg_ref[0]
    onehot_t = (jax.lax.broadcasted_iota(jnp.int32, (num_segments, TILE), 0)
                == seg).astype(jnp.bfloat16)
    acc_ref[...] += jnp.dot(onehot_t, h, preferred_element_type=jnp.float32)

    @pl.when(i == num_tiles - 1)
    def _finish():
        g = acc_ref[...].astype(jnp.bfloat16)
        z_mean = jnp.dot(g, wm_ref[...], preferred_element_type=jnp.float32) + bm_ref[...]
        z_var = jnp.dot(g, wv_ref[...], preferred_element_type=jnp.float32) + bv_ref[...]
        out_ref[...] = z_mean + jnp.exp(-0.5 * jnp.abs(z_var)) * eps_ref[...]


@jax.jit
def kernel(node_feats, segment_ids, W1, b1, W2, b2, Wm, bm, Wv, bv, eps):
    N, D = node_feats.shape
    H = W1.shape[1]
    L = Wm.shape[1]
    B = eps.shape[0]
    assert N % TILE == 0
    num_tiles = N // TILE

    seg3 = segment_ids.astype(jnp.int32).reshape(num_tiles, 1, TILE)

    in_specs = [
            pl.BlockSpec((TILE, D), lambda i: (i, 0)),
            pl.BlockSpec((1, 1, TILE), lambda i: (i, 0, 0)),
            pl.BlockSpec((D, H), lambda i: (0, 0)),
            pl.BlockSpec((1, H), lambda i: (0, 0)),
            pl.BlockSpec((H, D), lambda i: (0, 0)),
            pl.BlockSpec((1, D), lambda i: (0, 0)),
            pl.BlockSpec((D, L), lambda i: (0, 0)),
            pl.BlockSpec((1, L), lambda i: (0, 0)),
            pl.BlockSpec((D, L), lambda i: (0, 0)),
            pl.BlockSpec((1, L), lambda i: (0, 0)),
            pl.BlockSpec((B, L), lambda i: (0, 0)),
        ]

    return pl.pallas_call(
        partial(_fused_body, num_tiles=num_tiles, num_segments=B),
        grid=(num_tiles,),
        in_specs=in_specs,
        out_specs=pl.BlockSpec((B, L), lambda i: (0, 0)),
        out_shape=jax.ShapeDtypeStruct((B, L), jnp.float32),
        scratch_shapes=[pltpu.VMEM((B, D), jnp.float32)],
        compiler_params=pltpu.CompilerParams(
            dimension_semantics=("arbitrary",),
        ),
    )(node_feats.astype(jnp.bfloat16), seg3,
      W1.astype(jnp.bfloat16), b1.reshape(1, H),
      W2.astype(jnp.bfloat16), b2.reshape(1, D),
      Wm.astype(jnp.bfloat16), bm.reshape(1, L),
      Wv.astype(jnp.bfloat16), bv.reshape(1, L), eps)


# speedup vs baseline: 1.2485x; 1.2485x over previous
import jax
import jax.numpy as jnp
from jax.experimental import pallas as pl
from jax.experimental.pallas import tpu as pltpu
from functools import partial

TILE = 2048


def _fused_body(x_ref, seg_ref, w1_ref, b1_ref, w2_ref, b2_ref,
                wm_ref, bm_ref, wv_ref, bv_ref, eps_ref,
                out_ref, acc_ref, *, num_tiles, num_segments):
    i = pl.program_id(0)

    @pl.when(i == 0)
    def _init():
        acc_ref[...] = jnp.zeros_like(acc_ref)

    x = x_ref[...]
    h = jnp.dot(x, w1_ref[...], preferred_element_type=jnp.float32,
                precision=jax.lax.Precision.DEFAULT)
    h = jnp.maximum(h + b1_ref[...], 0.0)
    h = jnp.dot(h, w2_ref[...], preferred_element_type=jnp.float32,
                precision=jax.lax.Precision.DEFAULT)
    h = jnp.maximum(h + b2_ref[...], 0.0)

    seg = seg_ref[0]
    onehot_t = (jax.lax.broadcasted_iota(jnp.int32, (num_segments, TILE), 0)
                == seg).astype(jnp.float32)
    acc_ref[...] += jnp.dot(onehot_t, h, preferred_element_type=jnp.float32)

    @pl.when(i == num_tiles - 1)
    def _finish():
        g = acc_ref[...]
        z_mean = jnp.dot(g, wm_ref[...], preferred_element_type=jnp.float32) + bm_ref[...]
        z_var = jnp.dot(g, wv_ref[...], preferred_element_type=jnp.float32) + bv_ref[...]
        out_ref[...] = z_mean + jnp.exp(-0.5 * jnp.abs(z_var)) * eps_ref[...]


@jax.jit
def kernel(node_feats, segment_ids, W1, b1, W2, b2, Wm, bm, Wv, bv, eps):
    N, D = node_feats.shape
    H = W1.shape[1]
    L = Wm.shape[1]
    B = eps.shape[0]
    assert N % TILE == 0
    num_tiles = N // TILE

    seg3 = segment_ids.astype(jnp.int32).reshape(num_tiles, 1, TILE)

    in_specs = [
            pl.BlockSpec((TILE, D), lambda i: (i, 0)),
            pl.BlockSpec((1, 1, TILE), lambda i: (i, 0, 0)),
            pl.BlockSpec((D, H), lambda i: (0, 0)),
            pl.BlockSpec((1, H), lambda i: (0, 0)),
            pl.BlockSpec((H, D), lambda i: (0, 0)),
            pl.BlockSpec((1, D), lambda i: (0, 0)),
            pl.BlockSpec((D, L), lambda i: (0, 0)),
            pl.BlockSpec((1, L), lambda i: (0, 0)),
            pl.BlockSpec((D, L), lambda i: (0, 0)),
            pl.BlockSpec((1, L), lambda i: (0, 0)),
            pl.BlockSpec((B, L), lambda i: (0, 0)),
        ]

    return pl.pallas_call(
        partial(_fused_body, num_tiles=num_tiles, num_segments=B),
        grid=(num_tiles,),
        in_specs=in_specs,
        out_specs=pl.BlockSpec((B, L), lambda i: (0, 0)),
        out_shape=jax.ShapeDtypeStruct((B, L), jnp.float32),
        scratch_shapes=[pltpu.VMEM((B, D), jnp.float32)],
        compiler_params=pltpu.CompilerParams(
            dimension_semantics=("arbitrary",),
        ),
    )(node_feats, seg3, W1, b1.reshape(1, H), W2, b2.reshape(1, D),
      Wm, bm.reshape(1, L), Wv, bv.reshape(1, L), eps)
